# baseline (speedup 1.0000x reference)
import functools
import jax
import jax.numpy as jnp
from jax import lax
from jax.experimental import pallas as pl
from jax.experimental.pallas import tpu as pltpu
from jax.experimental.pallas import tpu_sc as plsc

VOCAB = 256
EMBED = 256
HIDDEN = 512
SEQ = 256
BATCH = 64

NW = 32
ROWS_PER_W = (SEQ * BATCH) // NW
CHUNK = 32
NCHUNK = ROWS_PER_W // CHUNK


def _table_kernel(emb_ref, wih_ref, bias_ref, table_ref):
    table_ref[...] = jnp.dot(emb_ref[...], wih_ref[...],
                             preferred_element_type=jnp.float32) + bias_ref[...]


def _make_table(emb, wih_t, bias):
    return pl.pallas_call(
        _table_kernel,
        out_shape=jax.ShapeDtypeStruct((VOCAB, 4 * HIDDEN), jnp.float32),
    )(emb, wih_t, bias)


def _sc_gather(table, idx):
    mesh = plsc.VectorSubcoreMesh(core_axis_name="c", subcore_axis_name="s")

    @functools.partial(
        pl.kernel, mesh=mesh,
        out_type=jax.ShapeDtypeStruct((SEQ * BATCH, 4 * HIDDEN), jnp.float32),
        scratch_types=[
            pltpu.VMEM((CHUNK,), jnp.int32),
            pltpu.VMEM((CHUNK, 4 * HIDDEN), jnp.float32),
            pltpu.SemaphoreType.DMA,
        ],
    )
    def gather(table_hbm, idx_hbm, out_hbm, idx_v, rows_v, sem):
        wid = lax.axis_index("s") * 2 + lax.axis_index("c")
        base = wid * ROWS_PER_W

        def body(j, carry):
            off = base + j * CHUNK
            pltpu.sync_copy(idx_hbm.at[pl.ds(off, CHUNK)], idx_v)
            pltpu.async_copy(table_hbm.at[idx_v], rows_v, sem).wait()
            pltpu.sync_copy(rows_v, out_hbm.at[pl.ds(off, CHUNK)])
            return carry

        lax.fori_loop(0, NCHUNK, body, 0)

    return gather(table, idx)


def _scan_kernel(inp_ref, whh_ref, wfc_ref, bfc_ref, out_ref, h_ref, c_ref):
    t = pl.program_id(0)

    @pl.when(t == 0)
    def _init():
        h_ref[...] = jnp.zeros_like(h_ref)
        c_ref[...] = jnp.zeros_like(c_ref)

    gates = inp_ref[0] + jnp.dot(h_ref[...].astype(jnp.bfloat16), whh_ref[...],
                                 preferred_element_type=jnp.float32)
    t4 = jnp.tanh(gates)
    i = 0.5 * t4[:, 0 * HIDDEN:1 * HIDDEN] + 0.5
    f = 0.5 * t4[:, 1 * HIDDEN:2 * HIDDEN] + 0.5
    g = t4[:, 2 * HIDDEN:3 * HIDDEN]
    o = 0.5 * t4[:, 3 * HIDDEN:4 * HIDDEN] + 0.5
    c_new = f * c_ref[...] + i * g
    h_new = o * jnp.tanh(c_new)
    c_ref[...] = c_new
    h_ref[...] = h_new

    @pl.when(t == SEQ - 1)
    def _fin():
        out_ref[...] = jnp.dot(h_new, wfc_ref[...],
                               preferred_element_type=jnp.float32) + bfc_ref[...]


def _scan(inp, whh_t, wfc_t, bfc):
    return pl.pallas_call(
        _scan_kernel,
        grid=(SEQ,),
        in_specs=[
            pl.BlockSpec((1, BATCH, 4 * HIDDEN), lambda t: (t, 0, 0)),
            pl.BlockSpec((HIDDEN, 4 * HIDDEN), lambda t: (0, 0)),
            pl.BlockSpec((HIDDEN, VOCAB), lambda t: (0, 0)),
            pl.BlockSpec((1, VOCAB), lambda t: (0, 0)),
        ],
        out_specs=pl.BlockSpec((BATCH, VOCAB), lambda t: (0, 0)),
        out_shape=jax.ShapeDtypeStruct((BATCH, VOCAB), jnp.float32),
        scratch_shapes=[
            pltpu.VMEM((BATCH, HIDDEN), jnp.float32),
            pltpu.VMEM((BATCH, HIDDEN), jnp.float32),
        ],
    )(inp, whh_t, wfc_t, bfc)


def kernel(x, emb, W_ih, W_hh, b_ih, b_hh, W_fc, b_fc):
    col = jax.lax.broadcasted_iota(jnp.int32, (1, 4 * HIDDEN), 1)
    scale = jnp.where((col >= 2 * HIDDEN) & (col < 3 * HIDDEN), 1.0, 0.5)
    wih_t = W_ih.T * scale
    whh_t = (W_hh.T * scale).astype(jnp.bfloat16)
    bias = (b_ih + b_hh).reshape(1, 4 * HIDDEN) * scale
    wfc_t = W_fc.T
    bfc = b_fc.reshape(1, VOCAB)

    table = _make_table(emb, wih_t, bias)
    idx = jnp.transpose(x.astype(jnp.int32), (1, 0)).reshape(SEQ * BATCH)
    inp = _sc_gather(table, idx).reshape(SEQ, BATCH, 4 * HIDDEN)
    return _scan(inp, whh_t, wfc_t, bfc)


# speedup vs baseline: 2.0854x; 2.0854x over previous
import jax
import jax.numpy as jnp
from jax.experimental import pallas as pl
from jax.experimental.pallas import tpu as pltpu

VOCAB = 256
EMBED = 256
HIDDEN = 512
SEQ = 256
BATCH = 64
G = 16
GROUP_ROWS = G * BATCH


def _lstm_kernel(xs_ref, emb_ref, wih_ref, whh_ref, bias_ref,
                 wfc_ref, bfc_ref, out_ref, table_ref, buf_ref):
    table_f32 = jnp.dot(emb_ref[...], wih_ref[...],
                        preferred_element_type=jnp.float32) + bias_ref[...]
    table_ref[...] = table_f32.astype(jnp.bfloat16)

    def lstm_step(inp, h, c):
        gates = inp + jnp.dot(h.astype(jnp.bfloat16), whh_ref[...],
                              preferred_element_type=jnp.float32)
        t4 = jnp.tanh(gates)
        i = 0.5 * t4[:, 0 * HIDDEN:1 * HIDDEN] + 0.5
        f = 0.5 * t4[:, 1 * HIDDEN:2 * HIDDEN] + 0.5
        g = t4[:, 2 * HIDDEN:3 * HIDDEN]
        o = 0.5 * t4[:, 3 * HIDDEN:4 * HIDDEN] + 0.5
        c_new = f * c + i * g
        h_new = o * jnp.tanh(c_new)
        return h_new, c_new

    def group(gi, carry):
        h, c = carry
        xg = xs_ref[pl.ds(gi * GROUP_ROWS, GROUP_ROWS), :]
        onehot = (xg == jax.lax.broadcasted_iota(
            jnp.int32, (GROUP_ROWS, VOCAB), 1)).astype(jnp.bfloat16)
        inp_g = jnp.dot(onehot, table_ref[...],
                        preferred_element_type=jnp.float32)
        buf_ref[...] = inp_g.astype(jnp.bfloat16)
        for k in range(G):
            inp = buf_ref[k * BATCH:(k + 1) * BATCH, :].astype(jnp.float32)
            h, c = lstm_step(inp, h, c)
        return h, c

    h0 = jnp.zeros((BATCH, HIDDEN), dtype=jnp.float32)
    c0 = jnp.zeros((BATCH, HIDDEN), dtype=jnp.float32)
    h_last, _ = jax.lax.fori_loop(0, SEQ // G, group, (h0, c0))
    out_ref[...] = jnp.dot(h_last, wfc_ref[...],
                           preferred_element_type=jnp.float32) + bfc_ref[...]


def kernel(x, emb, W_ih, W_hh, b_ih, b_hh, W_fc, b_fc):
    xs = jnp.transpose(x.astype(jnp.int32), (1, 0)).reshape(SEQ * BATCH, 1)
    col = jax.lax.broadcasted_iota(jnp.int32, (1, 4 * HIDDEN), 1)
    scale = jnp.where((col >= 2 * HIDDEN) & (col < 3 * HIDDEN), 1.0, 0.5)
    wih_t = W_ih.T * scale
    whh_t = (W_hh.T * scale).astype(jnp.bfloat16)
    bias = (b_ih + b_hh).reshape(1, 4 * HIDDEN) * scale
    wfc_t = W_fc.T
    bfc = b_fc.reshape(1, VOCAB)

    return pl.pallas_call(
        _lstm_kernel,
        out_shape=jax.ShapeDtypeStruct((BATCH, VOCAB), jnp.float32),
        scratch_shapes=[
            pltpu.VMEM((VOCAB, 4 * HIDDEN), jnp.bfloat16),
            pltpu.VMEM((GROUP_ROWS, 4 * HIDDEN), jnp.bfloat16),
        ],
    )(xs, emb, wih_t, whh_t, bias, wfc_t, bfc)
